# baseline (speedup 1.0000x reference)
import functools

import jax
import jax.numpy as jnp
from jax import lax
from jax.experimental import pallas as pl
from jax.experimental.pallas import tpu as pltpu
from jax.experimental.pallas import tpu_sc as plsc

NC = 2
NS = 16
NW = NC * NS
NBUF = 3


def _seq_embed_call(batch, seq_len, depth):
    bpw = batch // NW
    mesh = plsc.VectorSubcoreMesh(core_axis_name="c", subcore_axis_name="s")

    @functools.partial(
        pl.kernel,
        mesh=mesh,
        out_type=jax.ShapeDtypeStruct((batch, seq_len, depth), jnp.float32),
        scratch_types=[
            pltpu.VMEM((bpw * seq_len,), jnp.int32),
        ]
        + [pltpu.VMEM((seq_len, depth), jnp.float32) for _ in range(NBUF)]
        + [pltpu.SemaphoreType.DMA for _ in range(3 * NBUF)],
    )
    def run(seq_hbm, tok_hbm, pos_hbm, out_hbm, idx_v, *rest):
        bufs = rest[:NBUF]
        csems = rest[NBUF:2 * NBUF]
        gsems = rest[2 * NBUF:3 * NBUF]
        wsems = rest[3 * NBUF:]
        wid = lax.axis_index("s") * NC + lax.axis_index("c")
        b0 = wid * bpw
        pltpu.sync_copy(seq_hbm.at[wid], idx_v)

        def copy(s):
            return pltpu.make_async_copy(pos_hbm, bufs[s], csems[s])

        def gather(r, s):
            return pltpu.make_async_copy(
                tok_hbm.at[idx_v.at[pl.ds(r * seq_len, seq_len)]],
                bufs[s], gsems[s])

        def write(r, s):
            return pltpu.make_async_copy(bufs[s], out_hbm.at[b0 + r], wsems[s])

        for s in range(NBUF):
            copy(s).start()
        copy(0).wait()
        gather(0, 0).start(add=True)
        copy(1).wait()
        gather(1, 1).start(add=True)
        gather(0, 0).wait()
        write(0, 0).start()

        def group_body(i, c):
            for db in range(NBUF):
                r = 3 * i + 2 + db
                s = (2 + db) % NBUF
                copy(s).wait()
                gather(r, s).start(add=True)
                gather(r - 1, (s - 1) % NBUF).wait()
                write(r - 1, (s - 1) % NBUF).start()
                write(r - 2, (s + 1) % NBUF).wait()
                copy((s + 1) % NBUF).start()
            return c

        lax.fori_loop(0, (bpw - 5) // 3, group_body, 0)

        n = bpw
        for r in range(n - 3, n):
            s = r % NBUF
            copy(s).wait()
            gather(r, s).start(add=True)
            gather(r - 1, (s - 1) % NBUF).wait()
            write(r - 1, (s - 1) % NBUF).start()
            if r < n - 1:
                write(r - 2, (s + 1) % NBUF).wait()
                copy((s + 1) % NBUF).start()
        gather(n - 1, (n - 1) % NBUF).wait()
        write(n - 1, (n - 1) % NBUF).start()
        for r in range(n - 2, n + 1):
            write(r - 1, (r - 1) % NBUF).wait()

    return run


def kernel(seq, token_table, pos_table):
    batch, seq_len = seq.shape
    vocab, depth = token_table.shape
    bpw = batch // NW
    assert batch % NW == 0 and (bpw - 5) % 3 == 0 and bpw >= 8

    seq_perm = seq.reshape(NW, bpw * seq_len).astype(jnp.int32)

    return _seq_embed_call(batch, seq_len, depth)(
        seq_perm, token_table, pos_table)


# speedup vs baseline: 2.9948x; 2.9948x over previous
import functools

import jax
import jax.numpy as jnp
from jax import lax
from jax.experimental import pallas as pl
from jax.experimental.pallas import tpu as pltpu
from jax.experimental.pallas import tpu_sc as plsc

NC = 2
NS = 16
NW = NC * NS
LANES = 16
NBUF = 4


def _seq_embed_call(batch, seq_len, depth):
    bpw = batch // NW
    nvr = depth // LANES
    mesh = plsc.VectorSubcoreMesh(core_axis_name="c", subcore_axis_name="s")

    @functools.partial(
        pl.kernel,
        mesh=mesh,
        out_type=jax.ShapeDtypeStruct((batch, seq_len, depth), jnp.float32),
        scratch_types=[
            pltpu.VMEM((seq_len, bpw), jnp.int32),
            pltpu.VMEM((seq_len, depth), jnp.float32),
        ]
        + [pltpu.VMEM((bpw, depth), jnp.float32) for _ in range(NBUF)]
        + [pltpu.SemaphoreType.DMA for _ in range(2 * NBUF)],
    )
    def run(seq_hbm, tok_hbm, pos_hbm, out_hbm, idx_v, pos_v, *rest):
        bufs = rest[:NBUF]
        gsems = rest[NBUF:2 * NBUF]
        wsems = rest[2 * NBUF:]
        wid = lax.axis_index("s") * NC + lax.axis_index("c")
        b0 = wid * bpw
        pltpu.sync_copy(seq_hbm.at[wid], idx_v)
        pltpu.sync_copy(pos_hbm, pos_v)

        def gather(l, b):
            return pltpu.make_async_copy(
                tok_hbm.at[idx_v.at[l, :]], bufs[b], gsems[b])

        def write(l, b):
            return pltpu.make_async_copy(
                bufs[b], out_hbm.at[pl.ds(b0, bpw), l, :], wsems[b])

        def prefill(l, b):
            prow = [pos_v[l, pl.ds(k * LANES, LANES)] for k in range(nvr)]

            def row_body(r, c):
                for k in range(nvr):
                    bufs[b][r, pl.ds(k * LANES, LANES)] = prow[k]
                return c

            lax.fori_loop(0, bpw, row_body, 0)

        def step(l, b, wait_free):
            if wait_free:
                write(l - NBUF, b).wait()
            prefill(l, b)
            gather(l, b).start(add=True)
            gather(l - 1, (b - 1) % NBUF).wait()
            write(l - 1, (b - 1) % NBUF).start()

        prefill(0, 0)
        gather(0, 0).start(add=True)
        for l in range(1, NBUF):
            step(l, l, wait_free=False)

        def group_body(i, c):
            for db in range(NBUF):
                step(NBUF * i + db, db, wait_free=True)
            return c

        lax.fori_loop(1, seq_len // NBUF, group_body, 0)

        last = seq_len - 1
        gather(last, last % NBUF).wait()
        write(last, last % NBUF).start()
        for l in range(seq_len - NBUF, seq_len):
            write(l, l % NBUF).wait()

    return run


def kernel(seq, token_table, pos_table):
    batch, seq_len = seq.shape
    vocab, depth = token_table.shape
    bpw = batch // NW
    assert batch % NW == 0 and depth % LANES == 0 and seq_len % NBUF == 0

    seq_perm = jnp.transpose(
        seq.reshape(NW, bpw, seq_len).astype(jnp.int32), (0, 2, 1))

    return _seq_embed_call(batch, seq_len, depth)(
        seq_perm, token_table, pos_table)
